# baseline (device time: 108430 ns/iter reference)
import jax
import jax.numpy as jnp
from jax import lax
from jax.experimental import pallas as pl
from jax.experimental.pallas import tpu as pltpu

H_CHUNKS = 16


def kernel(A, B):
    m, k = A.shape
    k2, n = B.shape
    assert k == k2
    half = m // 2
    hc = half // H_CHUNKS

    def body(a_ref, b_ref, out_ref, send_raw, recv_raw, my_part, red,
             raw_send_sems, raw_recv_sems, red_send_sems, red_recv_sems,
             copy_sems):
        my_x = lax.axis_index("x")
        my_y = lax.axis_index("y")
        partner = (1 - my_x, my_y)
        m_off = my_x * half
        p_off = (1 - my_x) * half

        barrier = pltpu.get_barrier_semaphore()
        pl.semaphore_signal(
            barrier, inc=1, device_id=partner,
            device_id_type=pl.DeviceIdType.MESH,
        )
        pl.semaphore_wait(barrier, 1)

        b_bf16 = b_ref[...].astype(jnp.bfloat16)

        def raw_rdma(c):
            return pltpu.make_async_remote_copy(
                src_ref=send_raw.at[c],
                dst_ref=recv_raw.at[c],
                send_sem=raw_send_sems.at[c],
                recv_sem=raw_recv_sems.at[c],
                device_id=partner,
                device_id_type=pl.DeviceIdType.MESH,
            )

        def red_out_rdma(c, rows_off):
            return pltpu.make_async_remote_copy(
                src_ref=red.at[c],
                dst_ref=out_ref.at[pl.ds(rows_off + c * hc, hc), :],
                send_sem=red_send_sems.at[c],
                recv_sem=red_recv_sems.at[c],
                device_id=partner,
                device_id_type=pl.DeviceIdType.MESH,
            )

        for c in range(H_CHUNKS):
            send_raw[c] = jnp.dot(
                a_ref[pl.ds(p_off + c * hc, hc), :].astype(jnp.bfloat16),
                b_bf16,
                preferred_element_type=jnp.float32,
            ).astype(jnp.bfloat16)
            raw_rdma(c).start()

        for c in range(H_CHUNKS):
            my_part[c] = jnp.dot(
                a_ref[pl.ds(m_off + c * hc, hc), :].astype(jnp.bfloat16),
                b_bf16,
                preferred_element_type=jnp.float32,
            ).astype(jnp.bfloat16)

        for c in range(H_CHUNKS):
            rdma = raw_rdma(c)
            rdma.wait_recv()
            red[c] = (
                my_part[c].astype(jnp.float32)
                + recv_raw[c].astype(jnp.float32)
            ).astype(jnp.bfloat16)
            pltpu.make_async_copy(
                red.at[c],
                out_ref.at[pl.ds(m_off + c * hc, hc), :],
                copy_sems.at[c],
            ).start()
            red_out_rdma(c, m_off).start()
            rdma.wait_send()

        for c in range(H_CHUNKS):
            inbound = red_out_rdma(c, p_off)
            inbound.wait_recv()
            inbound.wait_send()
            pltpu.make_async_copy(
                red.at[c],
                out_ref.at[pl.ds(m_off + c * hc, hc), :],
                copy_sems.at[c],
            ).wait()

    return pl.pallas_call(
        body,
        out_shape=jax.ShapeDtypeStruct((m, n), jnp.bfloat16),
        in_specs=[
            pl.BlockSpec(memory_space=pltpu.VMEM),
            pl.BlockSpec(memory_space=pltpu.VMEM),
        ],
        out_specs=pl.BlockSpec(memory_space=pltpu.HBM),
        scratch_shapes=[
            pltpu.VMEM((H_CHUNKS, hc, n), jnp.bfloat16),
            pltpu.VMEM((H_CHUNKS, hc, n), jnp.bfloat16),
            pltpu.VMEM((H_CHUNKS, hc, n), jnp.bfloat16),
            pltpu.VMEM((H_CHUNKS, hc, n), jnp.bfloat16),
            pltpu.SemaphoreType.DMA((H_CHUNKS,)),
            pltpu.SemaphoreType.DMA((H_CHUNKS,)),
            pltpu.SemaphoreType.DMA((H_CHUNKS,)),
            pltpu.SemaphoreType.DMA((H_CHUNKS,)),
            pltpu.SemaphoreType.DMA((H_CHUNKS,)),
        ],
        compiler_params=pltpu.CompilerParams(
            collective_id=0,
            vmem_limit_bytes=100 * 1024 * 1024,
        ),
    )(A, B)


# device time: 108043 ns/iter; 1.0036x vs baseline; 1.0036x over previous
import jax
import jax.numpy as jnp
from jax import lax
from jax.experimental import pallas as pl
from jax.experimental.pallas import tpu as pltpu

H_CHUNKS = 8


def kernel(A, B):
    m, k = A.shape
    k2, n = B.shape
    assert k == k2
    half = m // 2
    hc = half // H_CHUNKS

    def body(a_ref, b_ref, out_ref, send_raw, recv_raw, my_part, red,
             raw_send_sems, raw_recv_sems, red_send_sems, red_recv_sems,
             copy_sems):
        my_x = lax.axis_index("x")
        my_y = lax.axis_index("y")
        partner = (1 - my_x, my_y)
        m_off = my_x * half
        p_off = (1 - my_x) * half

        barrier = pltpu.get_barrier_semaphore()
        pl.semaphore_signal(
            barrier, inc=1, device_id=partner,
            device_id_type=pl.DeviceIdType.MESH,
        )
        pl.semaphore_wait(barrier, 1)

        b_bf16 = b_ref[...].astype(jnp.bfloat16)

        def raw_rdma(c):
            return pltpu.make_async_remote_copy(
                src_ref=send_raw.at[c],
                dst_ref=recv_raw.at[c],
                send_sem=raw_send_sems.at[c],
                recv_sem=raw_recv_sems.at[c],
                device_id=partner,
                device_id_type=pl.DeviceIdType.MESH,
            )

        def red_out_rdma(c, rows_off):
            return pltpu.make_async_remote_copy(
                src_ref=red.at[c],
                dst_ref=out_ref.at[pl.ds(rows_off + c * hc, hc), :],
                send_sem=red_send_sems.at[c],
                recv_sem=red_recv_sems.at[c],
                device_id=partner,
                device_id_type=pl.DeviceIdType.MESH,
            )

        for c in range(H_CHUNKS):
            send_raw[c] = jnp.dot(
                a_ref[pl.ds(p_off + c * hc, hc), :].astype(jnp.bfloat16),
                b_bf16,
                preferred_element_type=jnp.float32,
            ).astype(jnp.bfloat16)
            raw_rdma(c).start()

        for c in range(H_CHUNKS):
            my_part[c] = jnp.dot(
                a_ref[pl.ds(m_off + c * hc, hc), :].astype(jnp.bfloat16),
                b_bf16,
                preferred_element_type=jnp.float32,
            ).astype(jnp.bfloat16)

        for c in range(H_CHUNKS):
            rdma = raw_rdma(c)
            rdma.wait_recv()
            red[c] = (
                my_part[c].astype(jnp.float32)
                + recv_raw[c].astype(jnp.float32)
            ).astype(jnp.bfloat16)
            pltpu.make_async_copy(
                red.at[c],
                out_ref.at[pl.ds(m_off + c * hc, hc), :],
                copy_sems.at[c],
            ).start()
            red_out_rdma(c, m_off).start()
            rdma.wait_send()

        for c in range(H_CHUNKS):
            inbound = red_out_rdma(c, p_off)
            inbound.wait_recv()
            inbound.wait_send()
            pltpu.make_async_copy(
                red.at[c],
                out_ref.at[pl.ds(m_off + c * hc, hc), :],
                copy_sems.at[c],
            ).wait()

    return pl.pallas_call(
        body,
        out_shape=jax.ShapeDtypeStruct((m, n), jnp.bfloat16),
        in_specs=[
            pl.BlockSpec(memory_space=pltpu.VMEM),
            pl.BlockSpec(memory_space=pltpu.VMEM),
        ],
        out_specs=pl.BlockSpec(memory_space=pltpu.HBM),
        scratch_shapes=[
            pltpu.VMEM((H_CHUNKS, hc, n), jnp.bfloat16),
            pltpu.VMEM((H_CHUNKS, hc, n), jnp.bfloat16),
            pltpu.VMEM((H_CHUNKS, hc, n), jnp.bfloat16),
            pltpu.VMEM((H_CHUNKS, hc, n), jnp.bfloat16),
            pltpu.SemaphoreType.DMA((H_CHUNKS,)),
            pltpu.SemaphoreType.DMA((H_CHUNKS,)),
            pltpu.SemaphoreType.DMA((H_CHUNKS,)),
            pltpu.SemaphoreType.DMA((H_CHUNKS,)),
            pltpu.SemaphoreType.DMA((H_CHUNKS,)),
        ],
        compiler_params=pltpu.CompilerParams(
            collective_id=0,
            vmem_limit_bytes=100 * 1024 * 1024,
        ),
    )(A, B)


# device time: 106819 ns/iter; 1.0151x vs baseline; 1.0115x over previous
import jax
import jax.numpy as jnp
from jax import lax
from jax.experimental import pallas as pl
from jax.experimental.pallas import tpu as pltpu

H_CHUNKS = 8


def kernel(A, B):
    m, k = A.shape
    k2, n = B.shape
    assert k == k2
    half = m // 2
    hc = half // H_CHUNKS

    def body(a_ref, b_ref, out_ref, a_vmem, send_raw, recv_raw, my_part, red,
             a_sems, raw_send_sems, raw_recv_sems, red_send_sems,
             red_recv_sems, copy_sems):
        my_x = lax.axis_index("x")
        my_y = lax.axis_index("y")
        partner = (1 - my_x, my_y)
        m_off = my_x * half
        p_off = (1 - my_x) * half

        def a_load(i, row_off):
            return pltpu.make_async_copy(
                a_ref.at[pl.ds(row_off, hc), :],
                a_vmem.at[i % 2],
                a_sems.at[i % 2],
            )

        def row_off(i):
            if i < H_CHUNKS:
                return p_off + i * hc
            return m_off + (i - H_CHUNKS) * hc

        a_load(0, row_off(0)).start()

        barrier = pltpu.get_barrier_semaphore()
        pl.semaphore_signal(
            barrier, inc=1, device_id=partner,
            device_id_type=pl.DeviceIdType.MESH,
        )
        pl.semaphore_wait(barrier, 1)

        b_bf16 = b_ref[...].astype(jnp.bfloat16)

        def raw_rdma(c):
            return pltpu.make_async_remote_copy(
                src_ref=send_raw.at[c],
                dst_ref=recv_raw.at[c],
                send_sem=raw_send_sems.at[c],
                recv_sem=raw_recv_sems.at[c],
                device_id=partner,
                device_id_type=pl.DeviceIdType.MESH,
            )

        def red_out_rdma(c, rows):
            return pltpu.make_async_remote_copy(
                src_ref=red.at[c],
                dst_ref=out_ref.at[pl.ds(rows + c * hc, hc), :],
                send_sem=red_send_sems.at[c],
                recv_sem=red_recv_sems.at[c],
                device_id=partner,
                device_id_type=pl.DeviceIdType.MESH,
            )

        for i in range(2 * H_CHUNKS):
            slot = i % 2
            a_load(i, row_off(i)).wait()
            if i + 1 < 2 * H_CHUNKS:
                a_load(i + 1, row_off(i + 1)).start()
            part = jnp.dot(
                a_vmem[slot].astype(jnp.bfloat16),
                b_bf16,
                preferred_element_type=jnp.float32,
            ).astype(jnp.bfloat16)
            if i < H_CHUNKS:
                send_raw[i] = part
                raw_rdma(i).start()
            else:
                my_part[i - H_CHUNKS] = part

        for c in range(H_CHUNKS):
            rdma = raw_rdma(c)
            rdma.wait_recv()
            red[c] = (
                my_part[c].astype(jnp.float32)
                + recv_raw[c].astype(jnp.float32)
            ).astype(jnp.bfloat16)
            pltpu.make_async_copy(
                red.at[c],
                out_ref.at[pl.ds(m_off + c * hc, hc), :],
                copy_sems.at[c],
            ).start()
            red_out_rdma(c, m_off).start()
            rdma.wait_send()

        for c in range(H_CHUNKS):
            inbound = red_out_rdma(c, p_off)
            inbound.wait_recv()
            inbound.wait_send()
            pltpu.make_async_copy(
                red.at[c],
                out_ref.at[pl.ds(m_off + c * hc, hc), :],
                copy_sems.at[c],
            ).wait()

    return pl.pallas_call(
        body,
        out_shape=jax.ShapeDtypeStruct((m, n), jnp.bfloat16),
        in_specs=[
            pl.BlockSpec(memory_space=pltpu.HBM),
            pl.BlockSpec(memory_space=pltpu.VMEM),
        ],
        out_specs=pl.BlockSpec(memory_space=pltpu.HBM),
        scratch_shapes=[
            pltpu.VMEM((2, hc, k), jnp.float32),
            pltpu.VMEM((H_CHUNKS, hc, n), jnp.bfloat16),
            pltpu.VMEM((H_CHUNKS, hc, n), jnp.bfloat16),
            pltpu.VMEM((H_CHUNKS, hc, n), jnp.bfloat16),
            pltpu.VMEM((H_CHUNKS, hc, n), jnp.bfloat16),
            pltpu.SemaphoreType.DMA((2,)),
            pltpu.SemaphoreType.DMA((H_CHUNKS,)),
            pltpu.SemaphoreType.DMA((H_CHUNKS,)),
            pltpu.SemaphoreType.DMA((H_CHUNKS,)),
            pltpu.SemaphoreType.DMA((H_CHUNKS,)),
            pltpu.SemaphoreType.DMA((H_CHUNKS,)),
        ],
        compiler_params=pltpu.CompilerParams(
            collective_id=0,
            vmem_limit_bytes=100 * 1024 * 1024,
        ),
    )(A, B)
